# Initial kernel scaffold; baseline (speedup 1.0000x reference)
import jax
import jax.numpy as jnp
from jax.experimental import pallas as pl


def kernel(edge_attr, W0, W1, W2):
    raise NotImplementedError("write your pallas kernel here")


# speedup vs baseline: 1.0921x; 1.0921x over previous
import functools
import jax
import jax.numpy as jnp
from jax import lax
from jax.experimental import pallas as pl
from jax.experimental.pallas import tpu as pltpu
from jax.experimental.pallas import tpu_sc as plsc

NUM_EDGES = 320000
EMB = 128

NC = 2
NS = 16
NW = NC * NS
EDGES_PER_W = NUM_EDGES // NW
CHUNK = 400
NCHUNK = EDGES_PER_W // CHUNK
GB = 80
NG = CHUNK // GB
CODE_VECS = CHUNK // 16


def _combine_body(t_ref, c_ref):
    row = lax.broadcasted_iota(jnp.int32, (60, 16), 0)
    col = lax.broadcasted_iota(jnp.int32, (60, 16), 1)
    m = ((col == row // 12).astype(jnp.float32)
         + (col == 5 + (row // 2) % 6).astype(jnp.float32)
         + (col == 11 + row % 2).astype(jnp.float32))
    c_ref[...] = jnp.dot(m, t_ref[...], preferred_element_type=jnp.float32)


def _build_combined(W0, W1, W2):
    t = jnp.zeros((16, EMB), jnp.float32)
    t = t.at[0:5].set(W0).at[5:11].set(W1).at[11:13].set(W2)
    return pl.pallas_call(
        _combine_body,
        out_shape=jax.ShapeDtypeStruct((60, EMB), jnp.float32),
    )(t)


def _sc_body(a0_hbm, a1_hbm, a2_hbm, c_hbm, out_hbm,
             a0_v0, a0_v1, a1_v0, a1_v1, a2_v0, a2_v1,
             codes_v, rows_v0, rows_v1, asem, gsem, osem0, osem1):
    wid = lax.axis_index("s") * NC + lax.axis_index("c")
    osems = (osem0, osem1)
    attr_bufs = ((a0_v0, a1_v0, a2_v0), (a0_v1, a1_v1, a2_v1))
    rows_bufs = (rows_v0, rows_v1)
    out_handles = [None, None]
    attr_handles = [None, None]

    def fetch_attr(ch, pb):
        base = wid * EDGES_PER_W + ch * CHUNK
        av0, av1, av2 = attr_bufs[pb]
        attr_handles[pb] = [
            pltpu.async_copy(a0_hbm.at[pl.ds(base, CHUNK)], av0, asem),
            pltpu.async_copy(a1_hbm.at[pl.ds(base, CHUNK)], av1, asem),
            pltpu.async_copy(a2_hbm.at[pl.ds(base, CHUNK)], av2, asem),
        ]

    fetch_attr(0, 0)

    for ch in range(NCHUNK):
        b = ch % 2
        base = wid * EDGES_PER_W + ch * CHUNK
        av0, av1, av2 = attr_bufs[b]
        rows_v = rows_bufs[b]

        for h in attr_handles[b]:
            h.wait()
        attr_handles[b] = None

        def code_body(i, _, av0=av0, av1=av1, av2=av2):
            s = pl.ds(i * 16, 16)
            va0 = jnp.clip(av0[s], 0, 4)
            va1 = jnp.clip(av1[s], 0, 5)
            va2 = jnp.clip(av2[s], 0, 1)
            codes_v[s] = (va0 * 6 + va1) * 2 + va2
            return 0

        lax.fori_loop(0, CODE_VECS, code_body, 0)

        ghandles = [
            pltpu.async_copy(
                c_hbm.at[codes_v.at[pl.ds(g * GB, GB)]],
                rows_v.at[pl.ds(g * GB, GB)],
                gsem,
            )
            for g in range(NG)
        ]
        if ch + 1 < NCHUNK:
            fetch_attr(ch + 1, 1 - b)
        if out_handles[1 - b] is not None:
            out_handles[1 - b].wait()
            out_handles[1 - b] = None
        for h in ghandles:
            h.wait()
        out_handles[b] = pltpu.async_copy(
            rows_v, out_hbm.at[pl.ds(base, CHUNK)], osems[b]
        )

    for b in range(2):
        if out_handles[b] is not None:
            out_handles[b].wait()


def _sc_gather(a0, a1, a2, c_table):
    mesh = plsc.VectorSubcoreMesh(core_axis_name="c", subcore_axis_name="s")
    f = functools.partial(
        pl.kernel,
        mesh=mesh,
        out_type=jax.ShapeDtypeStruct((NUM_EDGES, EMB), jnp.float32),
        scratch_types=[
            pltpu.VMEM((CHUNK,), jnp.int32),
            pltpu.VMEM((CHUNK,), jnp.int32),
            pltpu.VMEM((CHUNK,), jnp.int32),
            pltpu.VMEM((CHUNK,), jnp.int32),
            pltpu.VMEM((CHUNK,), jnp.int32),
            pltpu.VMEM((CHUNK,), jnp.int32),
            pltpu.VMEM((CHUNK,), jnp.int32),
            pltpu.VMEM((CHUNK, EMB), jnp.float32),
            pltpu.VMEM((CHUNK, EMB), jnp.float32),
            pltpu.SemaphoreType.DMA,
            pltpu.SemaphoreType.DMA,
            pltpu.SemaphoreType.DMA,
            pltpu.SemaphoreType.DMA,
        ],
    )(_sc_body)
    return f(a0, a1, a2, c_table)


def kernel(edge_attr, W0, W1, W2):
    c = _build_combined(W0, W1, W2)
    at = edge_attr.T
    return _sc_gather(at[0], at[1], at[2], c)
